# Initial kernel scaffold; baseline (speedup 1.0000x reference)
import jax
import jax.numpy as jnp
from jax.experimental import pallas as pl


def kernel(inputs, targets):
    raise NotImplementedError("write your pallas kernel here")


# speedup vs baseline: 15.0018x; 15.0018x over previous
import functools

import jax
import jax.numpy as jnp
from jax.experimental import pallas as pl
from jax.experimental.pallas import tpu as pltpu

NUM_CLASSES = 100000
ALPHA = 0.9
TOPRATIO = 0.01
BATCH = 1024
EPS = 1e-07

CLAMP_LO = -25.0
CLAMP_HI = 25.0
NT = 16
ROUNDS = 3
CBLK = 1024


def _body(x_ref, tgt_ref, out_ref, rng_ref, cnt_ref, fin_ref, *,
          nrows, ncols, cblk, ncb, k, rounds, nt, alpha):
    r = pl.program_id(0)
    cb = pl.program_id(1)
    kf = jnp.float32(k)

    @pl.when(jnp.logical_and(r == 0, cb == 0))
    def _():
        rng_ref[:, 0:1] = jnp.full((nrows, 1), CLAMP_LO, jnp.float32)
        rng_ref[:, 1:2] = jnp.full((nrows, 1), CLAMP_HI, jnp.float32)
        out_ref[...] = jnp.zeros((1, 1), jnp.float32)

    @pl.when(cb == 0)
    def _():
        cnt_ref[...] = jnp.zeros_like(cnt_ref)
        fin_ref[...] = jnp.zeros_like(fin_ref)

    x = x_ref[...]
    tgt = tgt_ref[...]
    cols = jax.lax.broadcasted_iota(jnp.int32, (nrows, cblk), 1) + cb * cblk
    valid = cols < ncols
    is_t = cols == tgt
    xc = jnp.where(is_t | ~valid, CLAMP_LO,
                   jnp.clip(x, CLAMP_LO, CLAMP_HI))
    lo = rng_ref[:, 0:1]
    hi = rng_ref[:, 1:2]

    @pl.when(r < rounds)
    def _():
        w = (hi - lo) * (1.0 / nt)
        for j in range(nt):
            t_j = lo + w * (j + 1)
            cnt_ref[:, j:j + 1] += jnp.sum(
                (xc > t_j).astype(jnp.float32), axis=1, keepdims=True)

    @pl.when(jnp.logical_and(r < rounds, cb == ncb - 1))
    def _():
        c = cnt_ref[...]
        jstar = jnp.sum((c > kf).astype(jnp.float32), axis=1, keepdims=True)
        jstar = jnp.minimum(jstar, jnp.float32(nt - 1))
        w = (hi - lo) * (1.0 / nt)
        rng_ref[:, 0:1] = lo + jstar * w
        rng_ref[:, 1:2] = lo + (jstar + 1.0) * w

    @pl.when(r == rounds)
    def _():
        probs = jax.nn.sigmoid(x)
        m = jnp.where(valid, -jnp.log(1.0 - probs + EPS), 0.0)
        above = (xc > hi).astype(jnp.float32)
        inbin = ((xc > lo) & (xc <= hi)).astype(jnp.float32)
        fin_ref[:, 0:1] += jnp.sum(m * above, axis=1, keepdims=True)
        fin_ref[:, 1:2] += jnp.sum(above, axis=1, keepdims=True)
        fin_ref[:, 2:3] += jnp.sum(m * inbin, axis=1, keepdims=True)
        fin_ref[:, 3:4] += jnp.sum(inbin, axis=1, keepdims=True)
        fin_ref[:, 4:5] += jnp.sum(
            jnp.where(is_t & valid, jnp.log(probs + EPS), 0.0),
            axis=1, keepdims=True)

    @pl.when(jnp.logical_and(r == rounds, cb == ncb - 1))
    def _():
        s_above = fin_ref[:, 0:1]
        a = fin_ref[:, 1:2]
        bin_sum = fin_ref[:, 2:3]
        bin_cnt = fin_ref[:, 3:4]
        pos = fin_ref[:, 4:5]
        rem = jnp.maximum(kf - a, 0.0)
        avg = bin_sum / jnp.maximum(bin_cnt, 1.0)
        s = s_above + rem * avg
        total = (-1.0 / nrows) * jnp.sum(pos) + \
                (alpha / (nrows * kf)) * jnp.sum(s)
        out_ref[...] = total.reshape(1, 1)


def _run(x, targets, *, k, rounds=ROUNDS, nt=NT, cblk=CBLK, alpha=ALPHA,
         interpret=False):
    nrows, ncols = x.shape
    ncb = (ncols + cblk - 1) // cblk
    tgt2 = targets.reshape(nrows, 1).astype(jnp.int32)
    body = functools.partial(_body, nrows=nrows, ncols=ncols, cblk=cblk,
                             ncb=ncb, k=k, rounds=rounds, nt=nt, alpha=alpha)
    out = pl.pallas_call(
        body,
        grid=(rounds + 1, ncb),
        in_specs=[
            pl.BlockSpec((nrows, cblk), lambda r, cb: (0, cb)),
            pl.BlockSpec((nrows, 1), lambda r, cb: (0, 0)),
        ],
        out_specs=pl.BlockSpec((1, 1), lambda r, cb: (0, 0)),
        out_shape=jax.ShapeDtypeStruct((1, 1), jnp.float32),
        scratch_shapes=[
            pltpu.VMEM((nrows, 2), jnp.float32),
            pltpu.VMEM((nrows, nt), jnp.float32),
            pltpu.VMEM((nrows, 8), jnp.float32),
        ],
        compiler_params=pltpu.CompilerParams(
            dimension_semantics=("arbitrary", "arbitrary")),
        interpret=interpret,
    )(x, tgt2)
    return out[0, 0]


def kernel(inputs, targets):
    k = int(NUM_CLASSES * TOPRATIO)
    return _run(inputs, targets, k=k)
